# Initial kernel scaffold; baseline (speedup 1.0000x reference)
import jax
import jax.numpy as jnp
from jax.experimental import pallas as pl


def kernel(x, W, b):
    raise NotImplementedError("write your pallas kernel here")


# speedup vs baseline: 1.6384x; 1.6384x over previous
import functools

import jax
import jax.numpy as jnp
from jax import lax
from jax.experimental import pallas as pl

_NE = 64
_K = 8
_EPS = 1e-6


def _gate_kernel(x_ref, w_ref, b_ref, out_ref):
    x = x_ref[...]
    w = w_ref[...]
    logits = lax.dot_general(w, x, (((1,), (1,)), ((), ())),
                             preferred_element_type=jnp.float32)
    logits = logits + b_ref[...]
    t = logits.shape[1]
    idx = lax.broadcasted_iota(jnp.int32, (_NE, t), 0)
    m = jnp.max(logits, axis=0, keepdims=True)
    e = jnp.exp(logits - m)
    z = jnp.sum(e, axis=0, keepdims=True)
    tv = jnp.full((1, t), jnp.inf, jnp.float32)
    ti = jnp.full((1, t), -1, jnp.int32)
    for _ in range(_K):
        elig = (logits < tv) | ((logits == tv) & (idx > ti))
        lm = jnp.where(elig, logits, -jnp.inf)
        tv = jnp.max(lm, axis=0, keepdims=True)
        ti = jnp.min(jnp.where(lm == tv, idx, _NE), axis=0, keepdims=True)
    mask = (logits > tv) | ((logits == tv) & (idx <= ti))
    es = jnp.where(mask, e, 0.0)
    s8 = jnp.sum(es, axis=0, keepdims=True)
    out_ref[...] = es / (s8 + _EPS * z)


@functools.partial(jax.jit, static_argnames=("block_t",))
def _switch_gate(x, w, b, block_t=512):
    bsz, seq, d = x.shape
    n_tok = bsz * seq
    xf = x.reshape(n_tok, d)
    grid = n_tok // block_t
    out_t = pl.pallas_call(
        _gate_kernel,
        grid=(grid,),
        in_specs=[
            pl.BlockSpec((block_t, d), lambda i: (i, 0)),
            pl.BlockSpec((_NE, d), lambda i: (0, 0)),
            pl.BlockSpec((_NE, 1), lambda i: (0, 0)),
        ],
        out_specs=pl.BlockSpec((_NE, block_t), lambda i: (0, i)),
        out_shape=jax.ShapeDtypeStruct((_NE, n_tok), jnp.float32),
    )(xf, w, b.reshape(_NE, 1))
    return out_t.T.reshape(bsz, seq, _NE)


def kernel(x, W, b):
    return _switch_gate(x, W, b)


# speedup vs baseline: 1.7720x; 1.0816x over previous
import functools

import jax
import jax.numpy as jnp
from jax import lax
from jax.experimental import pallas as pl

_NE = 64
_K = 8
_EPS = 1e-6


def _gate_kernel(x_ref, w_ref, b_ref, out_ref):
    x = x_ref[...]
    w = w_ref[...]
    logits = lax.dot_general(w, x, (((1,), (1,)), ((), ())),
                             preferred_element_type=jnp.float32)
    logits = logits + b_ref[...]
    t = logits.shape[1]
    idx = lax.broadcasted_iota(jnp.int32, (_NE, t), 0)
    m = jnp.max(logits, axis=0, keepdims=True)
    e = jnp.exp(logits - m)
    z = jnp.sum(e, axis=0, keepdims=True)
    tv = jnp.full((1, t), jnp.inf, jnp.float32)
    ti = jnp.full((1, t), -1, jnp.int32)
    for _ in range(_K):
        elig = (logits < tv) | ((logits == tv) & (idx > ti))
        lm = jnp.where(elig, logits, -jnp.inf)
        tv = jnp.max(lm, axis=0, keepdims=True)
        ti = jnp.min(jnp.where(lm == tv, idx, _NE), axis=0, keepdims=True)
    mask = (logits > tv) | ((logits == tv) & (idx <= ti))
    es = jnp.where(mask, e, 0.0)
    s8 = jnp.sum(es, axis=0, keepdims=True)
    out_ref[...] = es / (s8 + _EPS * z)


@functools.partial(jax.jit, static_argnames=("block_t",))
def _switch_gate(x, w, b, block_t=512):
    bsz, seq, d = x.shape
    n_tok = bsz * seq
    xf = x.reshape(n_tok, d)
    grid = n_tok // block_t
    out_t = pl.pallas_call(
        _gate_kernel,
        grid=(grid,),
        in_specs=[
            pl.BlockSpec((block_t, d), lambda i: (i, 0)),
            pl.BlockSpec((_NE, d), lambda i: (0, 0)),
            pl.BlockSpec((_NE, 1), lambda i: (0, 0)),
        ],
        out_specs=pl.BlockSpec((_NE, block_t), lambda i: (0, i)),
        out_shape=jax.ShapeDtypeStruct((_NE, n_tok), jnp.float32),
    )(xf, w, b.reshape(_NE, 1))
    return out_t.T.reshape(bsz, seq, _NE)


def kernel(x, W, b):
    return _switch_gate(x, W, b, block_t=1024)
